# Initial kernel scaffold; baseline (speedup 1.0000x reference)
import jax
import jax.numpy as jnp
from jax.experimental import pallas as pl


def kernel(token_ids, weight):
    raise NotImplementedError("write your pallas kernel here")


# speedup vs baseline: 1.1087x; 1.1087x over previous
import functools

import jax
import jax.numpy as jnp
from jax import lax
from jax.experimental import pallas as pl
from jax.experimental.pallas import tpu as pltpu
from jax.experimental.pallas import tpu_sc as plsc

EMBED_DIM = 32


@functools.lru_cache(maxsize=None)
def _make_gather(n_rows: int, vocab: int, chunk: int):
    info = plsc.get_sparse_core_info()
    nc, ns = info.num_cores, info.num_subcores
    nw = nc * ns
    rows_per_w = n_rows // nw
    assert rows_per_w % chunk == 0
    n_chunks = rows_per_w // chunk

    mesh = plsc.VectorSubcoreMesh(core_axis_name="c", subcore_axis_name="s")

    @functools.partial(
        pl.kernel,
        mesh=mesh,
        out_type=jax.ShapeDtypeStruct((n_rows, EMBED_DIM), jnp.float32),
        scratch_types=[
            pltpu.VMEM((chunk,), jnp.int32),
            pltpu.VMEM((chunk, EMBED_DIM), jnp.float32),
            pltpu.SemaphoreType.DMA,
        ],
        compiler_params=pltpu.CompilerParams(use_tc_tiling_on_sc=False),
    )
    def k(idx_hbm, table_hbm, out_hbm, idx_v, rows_v, sem):
        wid = lax.axis_index("s") * nc + lax.axis_index("c")
        base = wid * rows_per_w

        def body(i, carry):
            off = base + i * chunk
            pltpu.sync_copy(idx_hbm.at[pl.ds(off, chunk)], idx_v)
            pltpu.async_copy(table_hbm.at[idx_v], rows_v, sem).wait()
            pltpu.sync_copy(rows_v, out_hbm.at[pl.ds(off, chunk)])
            return carry

        lax.fori_loop(0, n_chunks, body, 0)

    return k


def kernel(token_ids, weight):
    batch, hist = token_ids.shape
    n_rows = batch * hist
    flat_idx = token_ids.reshape(n_rows).astype(jnp.int32)
    gather = _make_gather(n_rows, weight.shape[0], 2560)
    out = gather(flat_idx, weight)
    return out.reshape(batch, hist, EMBED_DIM)


# speedup vs baseline: 1.1095x; 1.0008x over previous
import functools

import jax
import jax.numpy as jnp
from jax import lax
from jax.experimental import pallas as pl
from jax.experimental.pallas import tpu as pltpu
from jax.experimental.pallas import tpu_sc as plsc

EMBED_DIM = 32
NBUF = 4
CHUNK = 640


@functools.lru_cache(maxsize=None)
def _make_gather(n_rows: int):
    info = plsc.get_sparse_core_info()
    nc, ns = info.num_cores, info.num_subcores
    nw = nc * ns
    rows_per_w = n_rows // nw
    assert rows_per_w % (CHUNK * NBUF) == 0
    n_outer = rows_per_w // (CHUNK * NBUF)

    mesh = plsc.VectorSubcoreMesh(core_axis_name="c", subcore_axis_name="s")

    @functools.partial(
        pl.kernel,
        mesh=mesh,
        out_type=jax.ShapeDtypeStruct((n_rows, EMBED_DIM), jnp.float32),
        scratch_types=[
            pltpu.VMEM((rows_per_w,), jnp.int32),
            pltpu.VMEM((NBUF, CHUNK, EMBED_DIM), jnp.float32),
            pltpu.SemaphoreType.DMA((NBUF,)),
            pltpu.SemaphoreType.DMA((NBUF,)),
        ],
        compiler_params=pltpu.CompilerParams(use_tc_tiling_on_sc=False),
    )
    def k(idx_hbm, table_hbm, out_hbm, idx_v, rows_v, gsem, ssem):
        wid = lax.axis_index("s") * nc + lax.axis_index("c")
        base = wid * rows_per_w
        pltpu.sync_copy(idx_hbm.at[pl.ds(base, rows_per_w)], idx_v)

        def outer(g, carry):
            goff = g * (CHUNK * NBUF)
            gathers = []
            for b in range(NBUF):
                coff = goff + b * CHUNK

                @pl.when(g > 0)
                def _drain(b=b):
                    pltpu.make_async_copy(
                        rows_v.at[b], out_hbm.at[pl.ds(base, CHUNK)], ssem.at[b]
                    ).wait()

                gathers.append(
                    pltpu.async_copy(
                        table_hbm.at[idx_v.at[pl.ds(coff, CHUNK)]],
                        rows_v.at[b],
                        gsem.at[b],
                    )
                )
            for b in range(NBUF):
                coff = goff + b * CHUNK
                gathers[b].wait()
                pltpu.async_copy(
                    rows_v.at[b], out_hbm.at[pl.ds(base + coff, CHUNK)], ssem.at[b]
                )
            return carry

        lax.fori_loop(0, n_outer, outer, 0)
        for b in range(NBUF):
            pltpu.make_async_copy(
                rows_v.at[b], out_hbm.at[pl.ds(base, CHUNK)], ssem.at[b]
            ).wait()

    return k


def kernel(token_ids, weight):
    batch, hist = token_ids.shape
    n_rows = batch * hist
    flat_idx = token_ids.reshape(n_rows).astype(jnp.int32)
    out = _make_gather(n_rows)(flat_idx, weight)
    return out.reshape(batch, hist, EMBED_DIM)


# speedup vs baseline: 1.4736x; 1.3281x over previous
import functools

import jax
import jax.numpy as jnp
from jax import lax
from jax.experimental import pallas as pl
from jax.experimental.pallas import tpu as pltpu
from jax.experimental.pallas import tpu_sc as plsc

D = 32
BW = 512
LANES = 16


@functools.lru_cache(maxsize=None)
def _make_gather(batch: int, hist: int):
    info = plsc.get_sparse_core_info()
    nc, ns = info.num_cores, info.num_subcores
    nw = nc * ns
    assert batch % (nw * BW) == 0 and batch // nw == BW
    n_btiles = BW // 128
    n_dtiles = D // 8
    h_stride = n_dtiles * (batch // 128) * 8 * 128
    dt_stride = (batch // 128) * 8 * 128
    out_words = hist * h_stride

    mesh = plsc.VectorSubcoreMesh(core_axis_name="c", subcore_axis_name="s")

    @functools.partial(
        pl.kernel,
        mesh=mesh,
        out_type=jax.ShapeDtypeStruct((out_words,), jnp.float32),
        scratch_types=[
            pltpu.VMEM((BW,), jnp.int32),
            pltpu.VMEM((BW, D), jnp.float32),
            pltpu.VMEM((BW * D,), jnp.float32),
            pltpu.SemaphoreType.DMA,
        ],
        compiler_params=pltpu.CompilerParams(
            use_tc_tiling_on_sc=False, needs_layout_passes=False
        ),
    )
    def k(idx_hbm, table_hbm, out_hbm, idx_v, rows_v, tiles_v, sem):
        wid = lax.axis_index("s") * nc + lax.axis_index("c")
        lane = lax.iota(jnp.int32, LANES)

        def per_h(h, carry):
            pltpu.sync_copy(idx_hbm.at[pl.ds(h * batch + wid * BW, BW)], idx_v)
            pltpu.async_copy(table_hbm.at[idx_v], rows_v, sem).wait()

            def per_seg(t, carry2):
                b0 = t * LANES
                dst_base = (t // 8) * 1024 + (t % 8) * LANES
                bvec = lane + b0
                for d in range(D):
                    vec = plsc.load_gather(
                        rows_v, [bvec, jnp.full((LANES,), d, jnp.int32)]
                    )
                    dt, di = d // 8, d % 8
                    tiles_v[pl.ds(dst_base + dt * 4096 + di * 128, LANES)] = vec
                return carry2

            lax.fori_loop(0, 32, per_seg, 0)

            out_base = h * h_stride + wid * (n_btiles * 8 * 128)
            for dt in range(n_dtiles):
                pltpu.async_copy(
                    tiles_v.at[pl.ds(dt * 4096, 4096)],
                    out_hbm.at[pl.ds(out_base + dt * dt_stride, 4096)],
                    sem,
                ).wait()
            return carry

        lax.fori_loop(0, hist, per_h, 0)

    return k


def kernel(token_ids, weight):
    batch, hist = token_ids.shape
    flat_idx = token_ids.T.reshape(batch * hist).astype(jnp.int32)
    outbuf = _make_gather(batch, hist)(flat_idx, weight)
    out = (
        outbuf.reshape(hist, D // 8, batch // 128, 8, 128)
        .transpose(2, 4, 0, 1, 3)
        .reshape(batch, hist, D)
    )
    return out


# speedup vs baseline: 1.6495x; 1.1194x over previous
import functools

import jax
import jax.numpy as jnp
from jax import lax
from jax.experimental import pallas as pl
from jax.experimental.pallas import tpu as pltpu
from jax.experimental.pallas import tpu_sc as plsc

D = 32
BW = 512
LANES = 16


@functools.lru_cache(maxsize=None)
def _make_gather(batch: int, hist: int):
    info = plsc.get_sparse_core_info()
    nc, ns = info.num_cores, info.num_subcores
    nw = nc * ns
    assert batch // nw == BW and hist % 2 == 0
    n_pairs = hist // 2
    n_dtiles = D // 8
    tile_words = BW // 128 * 8 * 128
    dt_stride = (batch // 128) * 8 * 128
    h_stride = n_dtiles * dt_stride
    out_words = hist * h_stride

    mesh = plsc.VectorSubcoreMesh(core_axis_name="c", subcore_axis_name="s")

    @functools.partial(
        pl.kernel,
        mesh=mesh,
        out_type=jax.ShapeDtypeStruct((out_words,), jnp.float32),
        scratch_types=[
            pltpu.VMEM((hist, BW), jnp.int32),
            pltpu.VMEM((BW, D), jnp.float32),
            pltpu.VMEM((BW, D), jnp.float32),
            pltpu.VMEM((tile_words * n_dtiles,), jnp.float32),
            pltpu.VMEM((tile_words * n_dtiles,), jnp.float32),
            pltpu.SemaphoreType.DMA,
            pltpu.SemaphoreType.DMA,
            pltpu.SemaphoreType.DMA,
            pltpu.SemaphoreType.DMA,
        ],
        compiler_params=pltpu.CompilerParams(
            use_tc_tiling_on_sc=False, needs_layout_passes=False
        ),
    )
    def k(idx_hbm, table_hbm, out_hbm, idx_v, rows_a, rows_b, tiles_a, tiles_b,
          gsa, gsb, ssa, ssb):
        wid = lax.axis_index("s") * nc + lax.axis_index("c")
        lane = lax.iota(jnp.int32, LANES)
        out_base = wid * tile_words

        pltpu.sync_copy(idx_hbm.at[pl.ds(0, hist), pl.ds(wid * BW, BW)], idx_v)
        pltpu.async_copy(table_hbm.at[idx_v.at[0]], rows_a, gsa)
        pltpu.async_copy(table_hbm.at[idx_v.at[1]], rows_b, gsb)

        def transpose(rows, tiles):
            def per_seg(t, c):
                dst_base = (t // 8) * 1024 + (t % 8) * LANES
                bvec = lane + t * LANES
                for d in range(D):
                    vec = plsc.load_gather(
                        rows, [bvec, jnp.full((LANES,), d, jnp.int32)]
                    )
                    tiles[pl.ds(dst_base + (d // 8) * 4096 + (d % 8) * 128,
                                LANES)] = vec
                return c

            lax.fori_loop(0, BW // LANES, per_seg, 0)

        def pair(j, carry):
            for s, rows, tiles, gs, ss in (
                (0, rows_a, tiles_a, gsa, ssa),
                (1, rows_b, tiles_b, gsb, ssb),
            ):
                h = 2 * j + s
                pltpu.make_async_copy(
                    table_hbm.at[idx_v.at[0]], rows, gs
                ).wait()

                @pl.when(j > 0)
                def _drain():
                    for dt in range(n_dtiles):
                        pltpu.make_async_copy(
                            tiles.at[pl.ds(dt * tile_words, tile_words)],
                            out_hbm.at[pl.ds(out_base, tile_words)],
                            ss,
                        ).wait()

                transpose(rows, tiles)
                for dt in range(n_dtiles):
                    pltpu.async_copy(
                        tiles.at[pl.ds(dt * tile_words, tile_words)],
                        out_hbm.at[
                            pl.ds(h * h_stride + dt * dt_stride + out_base,
                                  tile_words)
                        ],
                        ss,
                    )

                @pl.when(j < n_pairs - 1)
                def _next():
                    pltpu.async_copy(table_hbm.at[idx_v.at[h + 2]], rows, gs)

            return carry

        lax.fori_loop(0, n_pairs, pair, 0)
        for tiles, ss in ((tiles_a, ssa), (tiles_b, ssb)):
            for dt in range(n_dtiles):
                pltpu.make_async_copy(
                    tiles.at[pl.ds(dt * tile_words, tile_words)],
                    out_hbm.at[pl.ds(out_base, tile_words)],
                    ss,
                ).wait()

    return k


def kernel(token_ids, weight):
    batch, hist = token_ids.shape
    idx_t = token_ids.T.astype(jnp.int32)
    outbuf = _make_gather(batch, hist)(idx_t, weight)
    out = (
        outbuf.reshape(hist, D // 8, batch // 128, 8, 128)
        .transpose(2, 4, 0, 1, 3)
        .reshape(batch, hist, D)
    )
    return out


# speedup vs baseline: 1.9677x; 1.1929x over previous
import functools

import jax
import jax.numpy as jnp
from jax import lax
from jax.experimental import pallas as pl
from jax.experimental.pallas import tpu as pltpu
from jax.experimental.pallas import tpu_sc as plsc

D = 32
BW = 512
LANES = 16


@functools.lru_cache(maxsize=None)
def _make_gather(batch: int, hist: int):
    info = plsc.get_sparse_core_info()
    nc, ns = info.num_cores, info.num_subcores
    nw = nc * ns
    assert batch // nw == BW and hist % 2 == 0
    n_pairs = hist // 2
    n_dtiles = D // 8
    tile_words = BW // 128 * 8 * 128
    dt_stride = (batch // 128) * 8 * 128
    h_stride = n_dtiles * dt_stride
    out_words = hist * h_stride

    mesh = plsc.VectorSubcoreMesh(core_axis_name="c", subcore_axis_name="s")

    @functools.partial(
        pl.kernel,
        mesh=mesh,
        out_type=jax.ShapeDtypeStruct((out_words,), jnp.float32),
        scratch_types=[
            pltpu.VMEM((hist, BW), jnp.int32),
            pltpu.VMEM((BW, D), jnp.float32),
            pltpu.VMEM((BW, D), jnp.float32),
            pltpu.VMEM((tile_words * n_dtiles,), jnp.float32),
            pltpu.VMEM((tile_words * n_dtiles,), jnp.float32),
            pltpu.SemaphoreType.DMA,
            pltpu.SemaphoreType.DMA,
            pltpu.SemaphoreType.DMA,
            pltpu.SemaphoreType.DMA,
        ],
        compiler_params=pltpu.CompilerParams(
            use_tc_tiling_on_sc=False, needs_layout_passes=False
        ),
    )
    def k(idx_hbm, table_hbm, out_hbm, idx_v, rows_a, rows_b, tiles_a, tiles_b,
          gsa, gsb, ssa, ssb):
        wid = lax.axis_index("s") * nc + lax.axis_index("c")
        lane = lax.iota(jnp.int32, LANES)
        out_base = wid * tile_words

        pltpu.sync_copy(idx_hbm.at[pl.ds(0, hist), pl.ds(wid * BW, BW)], idx_v)
        pltpu.async_copy(table_hbm.at[idx_v.at[0]], rows_a, gsa)
        pltpu.async_copy(table_hbm.at[idx_v.at[1]], rows_b, gsb)

        def transpose(rows, tiles):
            @plsc.parallel_loop(0, BW // LANES, unroll=4)
            def per_seg(t):
                dst_base = (t // 8) * 1024 + (t % 8) * LANES
                bvec = lane + t * LANES
                for d in range(D):
                    vec = plsc.load_gather(
                        rows, [bvec, jnp.full((LANES,), d, jnp.int32)]
                    )
                    tiles[pl.ds(dst_base + (d // 8) * 4096 + (d % 8) * 128,
                                LANES)] = vec

        def pair(j, carry):
            for s, rows, tiles, gs, ss in (
                (0, rows_a, tiles_a, gsa, ssa),
                (1, rows_b, tiles_b, gsb, ssb),
            ):
                h = 2 * j + s
                pltpu.make_async_copy(
                    table_hbm.at[idx_v.at[0]], rows, gs
                ).wait()

                @pl.when(j > 0)
                def _drain():
                    for dt in range(n_dtiles):
                        pltpu.make_async_copy(
                            tiles.at[pl.ds(dt * tile_words, tile_words)],
                            out_hbm.at[pl.ds(out_base, tile_words)],
                            ss,
                        ).wait()

                transpose(rows, tiles)
                for dt in range(n_dtiles):
                    pltpu.async_copy(
                        tiles.at[pl.ds(dt * tile_words, tile_words)],
                        out_hbm.at[
                            pl.ds(h * h_stride + dt * dt_stride + out_base,
                                  tile_words)
                        ],
                        ss,
                    )

                @pl.when(j < n_pairs - 1)
                def _next():
                    pltpu.async_copy(table_hbm.at[idx_v.at[h + 2]], rows, gs)

            return carry

        lax.fori_loop(0, n_pairs, pair, 0)
        for tiles, ss in ((tiles_a, ssa), (tiles_b, ssb)):
            for dt in range(n_dtiles):
                pltpu.make_async_copy(
                    tiles.at[pl.ds(dt * tile_words, tile_words)],
                    out_hbm.at[pl.ds(out_base, tile_words)],
                    ss,
                ).wait()

    return k


def kernel(token_ids, weight):
    batch, hist = token_ids.shape
    idx_t = token_ids.T.astype(jnp.int32)
    outbuf = _make_gather(batch, hist)(idx_t, weight)
    out = (
        outbuf.reshape(hist, D // 8, batch // 128, 8, 128)
        .transpose(2, 4, 0, 1, 3)
        .reshape(batch, hist, D)
    )
    return out


# speedup vs baseline: 2.0398x; 1.0367x over previous
import functools

import jax
import jax.numpy as jnp
from jax import lax
from jax.experimental import pallas as pl
from jax.experimental.pallas import tpu as pltpu
from jax.experimental.pallas import tpu_sc as plsc

D = 32
BW = 512
LANES = 16


@functools.lru_cache(maxsize=None)
def _make_gather(batch: int, hist: int):
    info = plsc.get_sparse_core_info()
    nc, ns = info.num_cores, info.num_subcores
    nw = nc * ns
    assert batch // nw == BW and hist % 2 == 0
    n_pairs = hist // 2
    n_dtiles = D // 8
    tile_words = BW // 128 * 8 * 128
    dt_stride = (batch // 128) * 8 * 128
    h_stride = n_dtiles * dt_stride
    out_words = hist * h_stride

    mesh = plsc.VectorSubcoreMesh(core_axis_name="c", subcore_axis_name="s")

    @functools.partial(
        pl.kernel,
        mesh=mesh,
        out_type=jax.ShapeDtypeStruct((out_words,), jnp.float32),
        scratch_types=[
            pltpu.VMEM((BW * hist,), jnp.int32),
            pltpu.VMEM((BW,), jnp.int32),
            pltpu.VMEM((BW,), jnp.int32),
            pltpu.VMEM((BW, D), jnp.float32),
            pltpu.VMEM((BW, D), jnp.float32),
            pltpu.VMEM((tile_words * n_dtiles,), jnp.float32),
            pltpu.VMEM((tile_words * n_dtiles,), jnp.float32),
            pltpu.SemaphoreType.DMA,
            pltpu.SemaphoreType.DMA,
            pltpu.SemaphoreType.DMA,
            pltpu.SemaphoreType.DMA,
        ],
        compiler_params=pltpu.CompilerParams(
            use_tc_tiling_on_sc=False, needs_layout_passes=False
        ),
    )
    def k(idx_hbm, table_hbm, out_hbm, idx_blk, idx_a, idx_b, rows_a, rows_b,
          tiles_a, tiles_b, gsa, gsb, ssa, ssb):
        wid = lax.axis_index("s") * nc + lax.axis_index("c")
        lane = lax.iota(jnp.int32, LANES)
        lane_h = lane * hist
        out_base = wid * tile_words

        pltpu.sync_copy(idx_hbm.at[pl.ds(wid * (BW * hist), BW * hist)],
                        idx_blk)

        def repack(h, idxh):
            @plsc.parallel_loop(0, BW // LANES, unroll=4)
            def per_grp(g):
                vec = plsc.load_gather(idx_blk, [lane_h + (g * (LANES * hist) + h)])
                idxh[pl.ds(g * LANES, LANES)] = vec

        repack(0, idx_a)
        repack(1, idx_b)
        pltpu.async_copy(table_hbm.at[idx_a], rows_a, gsa)
        pltpu.async_copy(table_hbm.at[idx_b], rows_b, gsb)

        def transpose(rows, tiles):
            @plsc.parallel_loop(0, BW // LANES, unroll=4)
            def per_seg(t):
                dst_base = (t // 8) * 1024 + (t % 8) * LANES
                bvec = lane + t * LANES
                for d in range(D):
                    vec = plsc.load_gather(
                        rows, [bvec, jnp.full((LANES,), d, jnp.int32)]
                    )
                    tiles[pl.ds(dst_base + (d // 8) * 4096 + (d % 8) * 128,
                                LANES)] = vec

        def pair(j, carry):
            for s, idxh, rows, tiles, gs, ss in (
                (0, idx_a, rows_a, tiles_a, gsa, ssa),
                (1, idx_b, rows_b, tiles_b, gsb, ssb),
            ):
                h = 2 * j + s
                pltpu.make_async_copy(
                    table_hbm.at[idxh], rows, gs
                ).wait()

                @pl.when(j > 0)
                def _drain():
                    for dt in range(n_dtiles):
                        pltpu.make_async_copy(
                            tiles.at[pl.ds(dt * tile_words, tile_words)],
                            out_hbm.at[pl.ds(out_base, tile_words)],
                            ss,
                        ).wait()

                transpose(rows, tiles)
                for dt in range(n_dtiles):
                    pltpu.async_copy(
                        tiles.at[pl.ds(dt * tile_words, tile_words)],
                        out_hbm.at[
                            pl.ds(h * h_stride + dt * dt_stride + out_base,
                                  tile_words)
                        ],
                        ss,
                    )

                @pl.when(j < n_pairs - 1)
                def _next():
                    repack(h + 2, idxh)
                    pltpu.async_copy(table_hbm.at[idxh], rows, gs)

            return carry

        lax.fori_loop(0, n_pairs, pair, 0)
        for tiles, ss in ((tiles_a, ssa), (tiles_b, ssb)):
            for dt in range(n_dtiles):
                pltpu.make_async_copy(
                    tiles.at[pl.ds(dt * tile_words, tile_words)],
                    out_hbm.at[pl.ds(out_base, tile_words)],
                    ss,
                ).wait()

    return k


def kernel(token_ids, weight):
    batch, hist = token_ids.shape
    flat_idx = token_ids.reshape(batch * hist).astype(jnp.int32)
    outbuf = _make_gather(batch, hist)(flat_idx, weight)
    out = (
        outbuf.reshape(hist, D // 8, batch // 128, 8, 128)
        .transpose(2, 4, 0, 1, 3)
        .reshape(batch, hist, D)
    )
    return out
